# Initial kernel scaffold; baseline (speedup 1.0000x reference)
import jax
import jax.numpy as jnp
from jax.experimental import pallas as pl


def kernel(x, user_retweet_message_times, poc_att, edge_index, t_o):
    raise NotImplementedError("write your pallas kernel here")


# speedup vs baseline: 15.1137x; 15.1137x over previous
import functools

import jax
import jax.numpy as jnp
from jax import lax
from jax.experimental import pallas as pl
from jax.experimental.pallas import tpu as pltpu
from jax.experimental.pallas import tpu_sc as plsc

NC = 2
NS = 16
L = 16
NW = NC * NS

f32 = jnp.float32


def _mesh():
    return plsc.VectorSubcoreMesh(core_axis_name="c", subcore_axis_name="s",
                                  num_cores=NC, num_subcores=NS)


@functools.lru_cache(maxsize=None)
def _make_proj(n_pad, d):
    npt = n_pad // NW

    @functools.partial(
        pl.kernel,
        out_type=[jax.ShapeDtypeStruct((n_pad,), f32),
                  jax.ShapeDtypeStruct((n_pad,), f32)],
        mesh=_mesh(),
        compiler_params=pltpu.CompilerParams(needs_layout_passes=False),
        scratch_types=[
            pltpu.VMEM((d, npt), f32),
            pltpu.VMEM((d,), f32),
            pltpu.VMEM((d,), f32),
            pltpu.VMEM((npt,), f32),
            pltpu.VMEM((npt,), f32),
        ],
    )
    def proj(xt_hbm, a1_hbm, a2_hbm, s1_hbm, s2_hbm, xt_v, a1_v, a2_v, s1_v, s2_v):
        wid = lax.axis_index("s") * NC + lax.axis_index("c")
        base = wid * npt
        pltpu.sync_copy(xt_hbm.at[:, pl.ds(base, npt)], xt_v)
        pltpu.sync_copy(a1_hbm, a1_v)
        pltpu.sync_copy(a2_hbm, a2_v)
        a1c = [a1_v[pl.ds(r * L, L)] for r in range(d // L)]
        a2c = [a2_v[pl.ds(r * L, L)] for r in range(d // L)]

        def body(g, carry):
            sl = pl.ds(g * L, L)
            acc1 = jnp.zeros((L,), f32)
            acc2 = jnp.zeros((L,), f32)
            for dd in range(d):
                col = xt_v[dd, sl]
                acc1 = acc1 + col * a1c[dd // L][dd % L]
                acc2 = acc2 + col * a2c[dd // L][dd % L]
            s1_v[sl] = acc1
            s2_v[sl] = acc2
            return carry

        lax.fori_loop(0, npt // L, body, 0)
        pltpu.sync_copy(s1_v, s1_hbm.at[pl.ds(base, npt)])
        pltpu.sync_copy(s2_v, s2_hbm.at[pl.ds(base, npt)])

    return proj


@functools.lru_cache(maxsize=None)
def _make_edge(e, n_pad, n_acc):
    ept = e // NW

    @functools.partial(
        pl.kernel,
        out_type=[jax.ShapeDtypeStruct((e,), f32),
                  jax.ShapeDtypeStruct((NW, n_acc), f32)],
        mesh=_mesh(),
        compiler_params=pltpu.CompilerParams(needs_layout_passes=False),
        scratch_types=[
            pltpu.VMEM((ept,), jnp.int32),
            pltpu.VMEM((ept,), jnp.int32),
            pltpu.VMEM((ept,), f32),
            pltpu.VMEM((ept,), f32),
            pltpu.VMEM((n_pad,), f32),
            pltpu.VMEM((n_pad,), f32),
            pltpu.VMEM((n_acc,), f32),
            pltpu.VMEM((L,), f32),
        ],
    )
    def edge(src_hbm, dst_hbm, tt_hbm, s1_hbm, s2_hbm, t_hbm,
             p_hbm, dparts_hbm,
             src_v, dst_v, tt_v, p_v, s1_v, s2_v, den_v, t_v):
        wid = lax.axis_index("s") * NC + lax.axis_index("c")
        base = wid * ept
        pltpu.sync_copy(src_hbm.at[pl.ds(base, ept)], src_v)
        pltpu.sync_copy(dst_hbm.at[pl.ds(base, ept)], dst_v)
        pltpu.sync_copy(tt_hbm.at[pl.ds(base, ept)], tt_v)
        pltpu.sync_copy(s1_hbm, s1_v)
        pltpu.sync_copy(s2_hbm, s2_v)
        pltpu.sync_copy(t_hbm, t_v)

        zer = jnp.zeros((L,), f32)

        def zbody(i, carry):
            den_v[pl.ds(i * L, L)] = zer
            return carry

        lax.fori_loop(0, n_acc // L, zbody, 0)

        t_vec = t_v[...]

        def ebody(i, carry):
            sl = pl.ds(i * L, L)
            si = src_v[sl]
            di = dst_v[sl]
            tt = tt_v[sl]
            g = plsc.load_gather(s1_v, [si]) + plsc.load_gather(s2_v, [di])
            g = jnp.where(g > 0, g, 0.01 * g)
            val = g * jnp.exp(-2.0 * (t_vec - tt))
            pe = jnp.exp(-val)
            p_v[sl] = pe
            plsc.addupdate_scatter(den_v, [si], pe)
            return carry

        lax.fori_loop(0, ept // L, ebody, 0)
        pltpu.sync_copy(p_v, p_hbm.at[pl.ds(base, ept)])
        pltpu.sync_copy(den_v, dparts_hbm.at[wid])

    return edge


@functools.lru_cache(maxsize=None)
def _make_scat(e, n, n_acc, d):
    blk = 128
    tblk = e // blk
    assert e % blk == 0
    rpt = n_acc // NS
    assert rpt % blk == 0

    @functools.partial(
        pl.kernel,
        out_type=jax.ShapeDtypeStruct((NC, n_acc, d), f32),
        mesh=_mesh(),
        compiler_params=pltpu.CompilerParams(needs_layout_passes=False),
        scratch_types=[
            pltpu.VMEM_SHARED((n_acc, d), f32),
            pltpu.VMEM_SHARED((n_acc,), f32),
            pltpu.VMEM((rpt,), f32),
            pltpu.VMEM((n_acc,), f32),
            pltpu.VMEM((blk,), jnp.int32),
            pltpu.VMEM((blk,), jnp.int32),
            pltpu.VMEM((blk,), f32),
            pltpu.VMEM((blk,), f32),
            pltpu.VMEM((blk, d), f32),
            pltpu.SemaphoreType.DMA,
        ],
    )
    def scat(x_hbm, src_hbm, dst_hbm, p_hbm, dparts_hbm,
             parts_hbm,
             acc_sh, den_sh, strip_buf, den_v,
             sidx, didx, p_blk, alpha_v, rows_buf, sem):
        c = lax.axis_index("c")
        s = lax.axis_index("s")
        wid = s * NC + c
        colbase = s * rpt

        def dzero(k, carry):
            den_v[pl.ds(colbase + k * L, L)] = jnp.zeros((L,), f32)
            return carry

        lax.fori_loop(0, rpt // L, dzero, 0)

        def dpart(r, carry):
            pltpu.sync_copy(dparts_hbm.at[r, pl.ds(colbase, rpt)], strip_buf)

            def dadd(k, carry2):
                sl = pl.ds(colbase + k * L, L)
                den_v[sl] = den_v[sl] + strip_buf[pl.ds(k * L, L)]
                return carry2

            lax.fori_loop(0, rpt // L, dadd, 0)
            return carry

        lax.fori_loop(0, NW, dpart, 0)
        pltpu.sync_copy(den_v.at[pl.ds(colbase, rpt)],
                        den_sh.at[pl.ds(colbase, rpt)])

        zer = jnp.zeros((L,), f32)

        def zrow(i, carry):
            for r in range(d // L):
                rows_buf[i, pl.ds(r * L, L)] = zer
            return carry

        lax.fori_loop(0, blk, zrow, 0)

        def zacc(j, carry):
            pltpu.sync_copy(rows_buf, acc_sh.at[pl.ds(colbase + j * blk, blk)])
            return carry

        lax.fori_loop(0, rpt // blk, zacc, 0)
        plsc.subcore_barrier()
        pltpu.sync_copy(den_sh, den_v)

        nblk_i = tblk // NW + jnp.where(wid < tblk % NW, 1, 0)

        def mbody(j, carry):
            bb = (wid + j * NW) * blk
            pltpu.sync_copy(src_hbm.at[pl.ds(bb, blk)], sidx)
            pltpu.sync_copy(dst_hbm.at[pl.ds(bb, blk)], didx)
            pltpu.sync_copy(p_hbm.at[pl.ds(bb, blk)], p_blk)

            def abody(k, carry2):
                sl = pl.ds(k * L, L)
                si = sidx[sl]
                den16 = plsc.load_gather(den_v, [si])
                p16 = p_blk[sl]
                alpha_v[sl] = p16 / jnp.maximum(den16, 1e-12)
                return carry2

            lax.fori_loop(0, blk // L, abody, 0)
            pltpu.async_copy(x_hbm.at[didx], rows_buf, sem).wait()

            def sbody(ee, carry2):
                a16 = plsc.load_gather(alpha_v, [jnp.full((L,), ee, jnp.int32)])
                for r in range(d // L):
                    sl = pl.ds(r * L, L)
                    rows_buf[ee, sl] = rows_buf[ee, sl] * a16
                return carry2

            lax.fori_loop(0, blk, sbody, 0)
            pltpu.sync_copy(rows_buf, acc_sh.at[sidx], add=True)
            return carry

        lax.fori_loop(0, nblk_i, mbody, 0)
        plsc.subcore_barrier()

        def dmp(j, carry):
            rb = colbase + j * blk
            pltpu.sync_copy(acc_sh.at[pl.ds(rb, blk)], rows_buf)
            pltpu.sync_copy(rows_buf, parts_hbm.at[c, pl.ds(rb, blk)])
            return carry

        lax.fori_loop(0, rpt // blk, dmp, 0)

    return scat


@functools.lru_cache(maxsize=None)
def _make_comb(n_pad, d, rows):
    def body(p_ref, o_ref):
        v = p_ref[0] + p_ref[1]
        o_ref[...] = jnp.where(v > 0, v, jnp.exp(jnp.minimum(v, 0.0)) - 1.0)

    return pl.pallas_call(
        body,
        grid=(n_pad // rows,),
        in_specs=[pl.BlockSpec((NC, rows, d), lambda i: (0, i, 0))],
        out_specs=pl.BlockSpec((rows, d), lambda i: (i, 0)),
        out_shape=jax.ShapeDtypeStruct((n_pad, d), f32),
    )


def kernel(x, user_retweet_message_times, poc_att, edge_index, t_o):
    n, d = x.shape
    e = edge_index.shape[1]
    n_pad = ((n + NW * 128 - 1) // (NW * 128)) * (NW * 128)

    xt_pad = jnp.pad(x.T, ((0, 0), (0, n_pad - n)))
    a1 = poc_att[0, :d]
    a2 = poc_att[0, d:]
    t_vec = jnp.full((L,), t_o, dtype=f32)
    src = edge_index[0]
    dst = edge_index[1]
    tt = user_retweet_message_times.astype(f32)

    n_acc = ((n + NW * L - 1) // (NW * L)) * (NW * L)

    s1, s2 = _make_proj(n_pad, d)(xt_pad, a1, a2)
    p, dparts = _make_edge(e, n_pad, n_acc)(src, dst, tt, s1, s2, t_vec)
    parts = _make_scat(e, n, n_acc, d)(x, src, dst, p, dparts)
    out = _make_comb(n_acc, d, n_acc // 16)(parts)
    return out[:n]


# speedup vs baseline: 21.2016x; 1.4028x over previous
import functools

import jax
import jax.numpy as jnp
from jax import lax
from jax.experimental import pallas as pl
from jax.experimental.pallas import tpu as pltpu
from jax.experimental.pallas import tpu_sc as plsc

NC = 2
NS = 16
L = 16
NW = NC * NS

f32 = jnp.float32


def _mesh():
    return plsc.VectorSubcoreMesh(core_axis_name="c", subcore_axis_name="s",
                                  num_cores=NC, num_subcores=NS)


@functools.lru_cache(maxsize=None)
def _make_proj(n_pad, d):
    npt = n_pad // NW

    @functools.partial(
        pl.kernel,
        out_type=[jax.ShapeDtypeStruct((n_pad,), f32),
                  jax.ShapeDtypeStruct((n_pad,), f32)],
        mesh=_mesh(),
        compiler_params=pltpu.CompilerParams(needs_layout_passes=False),
        scratch_types=[
            pltpu.VMEM((d, npt), f32),
            pltpu.VMEM((d,), f32),
            pltpu.VMEM((d,), f32),
            pltpu.VMEM((npt,), f32),
            pltpu.VMEM((npt,), f32),
        ],
    )
    def proj(xt_hbm, a1_hbm, a2_hbm, s1_hbm, s2_hbm, xt_v, a1_v, a2_v, s1_v, s2_v):
        wid = lax.axis_index("s") * NC + lax.axis_index("c")
        base = wid * npt
        pltpu.sync_copy(xt_hbm.at[:, pl.ds(base, npt)], xt_v)
        pltpu.sync_copy(a1_hbm, a1_v)
        pltpu.sync_copy(a2_hbm, a2_v)
        a1c = [a1_v[pl.ds(r * L, L)] for r in range(d // L)]
        a2c = [a2_v[pl.ds(r * L, L)] for r in range(d // L)]

        def body(g, carry):
            sl = pl.ds(g * L, L)
            acc1 = jnp.zeros((L,), f32)
            acc2 = jnp.zeros((L,), f32)
            for dd in range(d):
                col = xt_v[dd, sl]
                acc1 = acc1 + col * a1c[dd // L][dd % L]
                acc2 = acc2 + col * a2c[dd // L][dd % L]
            s1_v[sl] = acc1
            s2_v[sl] = acc2
            return carry

        lax.fori_loop(0, npt // L, body, 0)
        pltpu.sync_copy(s1_v, s1_hbm.at[pl.ds(base, npt)])
        pltpu.sync_copy(s2_v, s2_hbm.at[pl.ds(base, npt)])

    return proj


@functools.lru_cache(maxsize=None)
def _make_edge(e, n_pad, n_acc):
    ept = e // NW

    @functools.partial(
        pl.kernel,
        out_type=[jax.ShapeDtypeStruct((e,), f32),
                  jax.ShapeDtypeStruct((NW, n_acc), f32)],
        mesh=_mesh(),
        compiler_params=pltpu.CompilerParams(needs_layout_passes=False),
        scratch_types=[
            pltpu.VMEM((ept,), jnp.int32),
            pltpu.VMEM((ept,), jnp.int32),
            pltpu.VMEM((ept,), f32),
            pltpu.VMEM((ept,), f32),
            pltpu.VMEM((n_pad,), f32),
            pltpu.VMEM((n_pad,), f32),
            pltpu.VMEM((n_acc,), f32),
            pltpu.VMEM((L,), f32),
        ],
    )
    def edge(src_hbm, dst_hbm, tt_hbm, s1_hbm, s2_hbm, t_hbm,
             p_hbm, dparts_hbm,
             src_v, dst_v, tt_v, p_v, s1_v, s2_v, den_v, t_v):
        wid = lax.axis_index("s") * NC + lax.axis_index("c")
        base = wid * ept
        pltpu.sync_copy(src_hbm.at[pl.ds(base, ept)], src_v)
        pltpu.sync_copy(dst_hbm.at[pl.ds(base, ept)], dst_v)
        pltpu.sync_copy(tt_hbm.at[pl.ds(base, ept)], tt_v)
        pltpu.sync_copy(s1_hbm, s1_v)
        pltpu.sync_copy(s2_hbm, s2_v)
        pltpu.sync_copy(t_hbm, t_v)

        zer = jnp.zeros((L,), f32)

        def zbody(i, carry):
            den_v[pl.ds(i * L, L)] = zer
            return carry

        lax.fori_loop(0, n_acc // L, zbody, 0)

        t_vec = t_v[...]

        def ebody(i, carry):
            sl = pl.ds(i * L, L)
            si = src_v[sl]
            di = dst_v[sl]
            tt = tt_v[sl]
            g = plsc.load_gather(s1_v, [si]) + plsc.load_gather(s2_v, [di])
            g = jnp.where(g > 0, g, 0.01 * g)
            val = g * jnp.exp(-2.0 * (t_vec - tt))
            pe = jnp.exp(-val)
            p_v[sl] = pe
            plsc.addupdate_scatter(den_v, [si], pe)
            return carry

        lax.fori_loop(0, ept // L, ebody, 0)
        pltpu.sync_copy(p_v, p_hbm.at[pl.ds(base, ept)])
        pltpu.sync_copy(den_v, dparts_hbm.at[wid])

    return edge


@functools.lru_cache(maxsize=None)
def _make_scat(e, n, n_acc, d):
    blk = 128
    tblk = e // blk
    assert e % (2 * blk) == 0
    tpair = tblk // 2
    rpt = n_acc // NS
    assert rpt % blk == 0

    @functools.partial(
        pl.kernel,
        out_type=jax.ShapeDtypeStruct((NC, n_acc, d), f32),
        mesh=_mesh(),
        compiler_params=pltpu.CompilerParams(needs_layout_passes=False),
        scratch_types=[
            pltpu.VMEM_SHARED((n_acc, d), f32),
            pltpu.VMEM_SHARED((n_acc,), f32),
            pltpu.VMEM((rpt,), f32),
            pltpu.VMEM((n_acc,), f32),
            [pltpu.VMEM((blk,), jnp.int32)] * 2,
            [pltpu.VMEM((blk,), jnp.int32)] * 2,
            [pltpu.VMEM((blk,), f32)] * 2,
            [pltpu.VMEM((blk,), f32)] * 2,
            [pltpu.VMEM((blk, d), f32)] * 2,
            pltpu.SemaphoreType.DMA,
            [pltpu.SemaphoreType.DMA] * 2,
            [pltpu.SemaphoreType.DMA] * 2,
        ],
    )
    def scat(x_hbm, src_hbm, dst_hbm, p_hbm, dparts_hbm,
             parts_hbm,
             acc_sh, den_sh, strip_buf, den_v,
             sidx2, didx2, p_blk2, alpha2, rows2, isem, gsem2, ssem2):
        sidx, didx, p_blk, alpha_v, rows_buf = (
            sidx2[0], didx2[0], p_blk2[0], alpha2[0], rows2[0])
        c = lax.axis_index("c")
        s = lax.axis_index("s")
        wid = s * NC + c
        colbase = s * rpt

        def dzero(k, carry):
            den_v[pl.ds(colbase + k * L, L)] = jnp.zeros((L,), f32)
            return carry

        lax.fori_loop(0, rpt // L, dzero, 0)

        def dpart(r, carry):
            pltpu.sync_copy(dparts_hbm.at[r, pl.ds(colbase, rpt)], strip_buf)

            def dadd(k, carry2):
                sl = pl.ds(colbase + k * L, L)
                den_v[sl] = den_v[sl] + strip_buf[pl.ds(k * L, L)]
                return carry2

            lax.fori_loop(0, rpt // L, dadd, 0)
            return carry

        lax.fori_loop(0, NW, dpart, 0)
        pltpu.sync_copy(den_v.at[pl.ds(colbase, rpt)],
                        den_sh.at[pl.ds(colbase, rpt)])

        zer = jnp.zeros((L,), f32)

        def zrow(i, carry):
            for r in range(d // L):
                rows_buf[i, pl.ds(r * L, L)] = zer
            return carry

        lax.fori_loop(0, blk, zrow, 0)

        def zacc(j, carry):
            pltpu.sync_copy(rows_buf, acc_sh.at[pl.ds(colbase + j * blk, blk)])
            return carry

        lax.fori_loop(0, rpt // blk, zacc, 0)
        plsc.subcore_barrier()
        pltpu.sync_copy(den_sh, den_v)

        npair_i = tpair // NW + jnp.where(wid < tpair % NW, 1, 0)

        def fetch_idx(pj, h):
            bb = pj * 2 * blk + h * blk
            return (
                pltpu.async_copy(src_hbm.at[pl.ds(bb, blk)], sidx2[h], isem),
                pltpu.async_copy(dst_hbm.at[pl.ds(bb, blk)], didx2[h], isem),
                pltpu.async_copy(p_hbm.at[pl.ds(bb, blk)], p_blk2[h], isem),
            )

        def calc_alpha(h):
            def abody(k, carry2):
                sl = pl.ds(k * L, L)
                den16 = plsc.load_gather(den_v, [sidx2[h][sl]])
                alpha2[h][sl] = p_blk2[h][sl] / jnp.maximum(den16, 1e-12)
                return carry2

            lax.fori_loop(0, blk // L, abody, 0)

        def scale_rows(h):
            def sbody(ee, carry2):
                a16 = plsc.load_gather(alpha2[h],
                                       [jnp.full((L,), ee, jnp.int32)])
                for r in range(d // L):
                    sl = pl.ds(r * L, L)
                    rows2[h][ee, sl] = rows2[h][ee, sl] * a16
                return carry2

            lax.fori_loop(0, blk, sbody, 0)

        def mbody(jj, carry):
            pj = wid + jj * NW
            da = fetch_idx(pj, 0)
            db = fetch_idx(pj, 1)
            for dsc in da:
                dsc.wait()
            calc_alpha(0)
            ga = pltpu.async_copy(x_hbm.at[didx2[0]], rows2[0], gsem2[0])
            for dsc in db:
                dsc.wait()
            calc_alpha(1)
            gb = pltpu.async_copy(x_hbm.at[didx2[1]], rows2[1], gsem2[1])
            ga.wait()
            scale_rows(0)
            sa = pltpu.async_copy(rows2[0], acc_sh.at[sidx2[0]], ssem2[0],
                                  add=True)
            gb.wait()
            scale_rows(1)
            sb = pltpu.async_copy(rows2[1], acc_sh.at[sidx2[1]], ssem2[1],
                                  add=True)
            sa.wait()
            sb.wait()
            return carry

        lax.fori_loop(0, npair_i, mbody, 0)
        plsc.subcore_barrier()

        def dmp(j, carry):
            rb = colbase + j * blk
            pltpu.sync_copy(acc_sh.at[pl.ds(rb, blk)], rows_buf)
            pltpu.sync_copy(rows_buf, parts_hbm.at[c, pl.ds(rb, blk)])
            return carry

        lax.fori_loop(0, rpt // blk, dmp, 0)

    return scat


@functools.lru_cache(maxsize=None)
def _make_comb(n_pad, d, rows):
    def body(p_ref, o_ref):
        v = p_ref[0] + p_ref[1]
        o_ref[...] = jnp.where(v > 0, v, jnp.exp(jnp.minimum(v, 0.0)) - 1.0)

    return pl.pallas_call(
        body,
        grid=(n_pad // rows,),
        in_specs=[pl.BlockSpec((NC, rows, d), lambda i: (0, i, 0))],
        out_specs=pl.BlockSpec((rows, d), lambda i: (i, 0)),
        out_shape=jax.ShapeDtypeStruct((n_pad, d), f32),
    )


def kernel(x, user_retweet_message_times, poc_att, edge_index, t_o):
    n, d = x.shape
    e = edge_index.shape[1]
    n_pad = ((n + NW * 128 - 1) // (NW * 128)) * (NW * 128)

    xt_pad = jnp.pad(x.T, ((0, 0), (0, n_pad - n)))
    a1 = poc_att[0, :d]
    a2 = poc_att[0, d:]
    t_vec = jnp.full((L,), t_o, dtype=f32)
    src = edge_index[0]
    dst = edge_index[1]
    tt = user_retweet_message_times.astype(f32)

    n_acc = ((n + NW * L - 1) // (NW * L)) * (NW * L)

    s1, s2 = _make_proj(n_pad, d)(xt_pad, a1, a2)
    p, dparts = _make_edge(e, n_pad, n_acc)(src, dst, tt, s1, s2, t_vec)
    parts = _make_scat(e, n, n_acc, d)(x, src, dst, p, dparts)
    out = _make_comb(n_acc, d, n_acc // 16)(parts)
    return out[:n]


# speedup vs baseline: 25.4058x; 1.1983x over previous
import functools

import jax
import jax.numpy as jnp
from jax import lax
from jax.experimental import pallas as pl
from jax.experimental.pallas import tpu as pltpu
from jax.experimental.pallas import tpu_sc as plsc

NC = 2
NS = 16
L = 16
NW = NC * NS

f32 = jnp.float32


def _mesh():
    return plsc.VectorSubcoreMesh(core_axis_name="c", subcore_axis_name="s",
                                  num_cores=NC, num_subcores=NS)


@functools.lru_cache(maxsize=None)
def _make_proj(n_pad, d):
    npt = n_pad // NW

    @functools.partial(
        pl.kernel,
        out_type=[jax.ShapeDtypeStruct((n_pad,), f32),
                  jax.ShapeDtypeStruct((n_pad,), f32)],
        mesh=_mesh(),
        compiler_params=pltpu.CompilerParams(needs_layout_passes=False),
        scratch_types=[
            pltpu.VMEM((d, npt), f32),
            pltpu.VMEM((d,), f32),
            pltpu.VMEM((d,), f32),
            pltpu.VMEM((npt,), f32),
            pltpu.VMEM((npt,), f32),
        ],
    )
    def proj(xt_hbm, a1_hbm, a2_hbm, s1_hbm, s2_hbm, xt_v, a1_v, a2_v, s1_v, s2_v):
        wid = lax.axis_index("s") * NC + lax.axis_index("c")
        base = wid * npt
        pltpu.sync_copy(xt_hbm.at[:, pl.ds(base, npt)], xt_v)
        pltpu.sync_copy(a1_hbm, a1_v)
        pltpu.sync_copy(a2_hbm, a2_v)
        a1c = [a1_v[pl.ds(r * L, L)] for r in range(d // L)]
        a2c = [a2_v[pl.ds(r * L, L)] for r in range(d // L)]

        def body(g, carry):
            sl = pl.ds(g * L, L)
            acc1 = jnp.zeros((L,), f32)
            acc2 = jnp.zeros((L,), f32)
            for dd in range(d):
                col = xt_v[dd, sl]
                acc1 = acc1 + col * a1c[dd // L][dd % L]
                acc2 = acc2 + col * a2c[dd // L][dd % L]
            s1_v[sl] = acc1
            s2_v[sl] = acc2
            return carry

        lax.fori_loop(0, npt // L, body, 0)
        pltpu.sync_copy(s1_v, s1_hbm.at[pl.ds(base, npt)])
        pltpu.sync_copy(s2_v, s2_hbm.at[pl.ds(base, npt)])

    return proj


@functools.lru_cache(maxsize=None)
def _make_edge(e, n_pad, n_acc):
    ept = e // NW

    @functools.partial(
        pl.kernel,
        out_type=[jax.ShapeDtypeStruct((e,), f32),
                  jax.ShapeDtypeStruct((NW, n_acc), f32)],
        mesh=_mesh(),
        compiler_params=pltpu.CompilerParams(needs_layout_passes=False),
        scratch_types=[
            pltpu.VMEM((ept,), jnp.int32),
            pltpu.VMEM((ept,), jnp.int32),
            pltpu.VMEM((ept,), f32),
            pltpu.VMEM((ept,), f32),
            pltpu.VMEM((n_pad,), f32),
            pltpu.VMEM((n_pad,), f32),
            pltpu.VMEM((n_acc,), f32),
            pltpu.VMEM((L,), f32),
        ],
    )
    def edge(src_hbm, dst_hbm, tt_hbm, s1_hbm, s2_hbm, t_hbm,
             p_hbm, dparts_hbm,
             src_v, dst_v, tt_v, p_v, s1_v, s2_v, den_v, t_v):
        wid = lax.axis_index("s") * NC + lax.axis_index("c")
        base = wid * ept
        pltpu.sync_copy(src_hbm.at[pl.ds(base, ept)], src_v)
        pltpu.sync_copy(dst_hbm.at[pl.ds(base, ept)], dst_v)
        pltpu.sync_copy(tt_hbm.at[pl.ds(base, ept)], tt_v)
        pltpu.sync_copy(s1_hbm, s1_v)
        pltpu.sync_copy(s2_hbm, s2_v)
        pltpu.sync_copy(t_hbm, t_v)

        zer = jnp.zeros((L,), f32)

        def zbody(i, carry):
            den_v[pl.ds(i * L, L)] = zer
            return carry

        lax.fori_loop(0, n_acc // L, zbody, 0)

        t_vec = t_v[...]

        def ebody(i, carry):
            sl = pl.ds(i * L, L)
            si = src_v[sl]
            di = dst_v[sl]
            tt = tt_v[sl]
            g = plsc.load_gather(s1_v, [si]) + plsc.load_gather(s2_v, [di])
            g = jnp.where(g > 0, g, 0.01 * g)
            val = g * jnp.exp(-2.0 * (t_vec - tt))
            pe = jnp.exp(-val)
            p_v[sl] = pe
            plsc.addupdate_scatter(den_v, [si], pe)
            return carry

        lax.fori_loop(0, ept // L, ebody, 0)
        pltpu.sync_copy(p_v, p_hbm.at[pl.ds(base, ept)])
        pltpu.sync_copy(den_v, dparts_hbm.at[wid])

    return edge


@functools.lru_cache(maxsize=None)
def _make_scat(e, n, n_acc, d):
    blk = 128
    tblk = e // blk
    assert e % (2 * blk) == 0
    tpair = tblk // 2
    rpt = n_acc // NS
    assert rpt % blk == 0

    @functools.partial(
        pl.kernel,
        out_type=jax.ShapeDtypeStruct((NC, n_acc, d), f32),
        mesh=_mesh(),
        compiler_params=pltpu.CompilerParams(needs_layout_passes=False),
        scratch_types=[
            pltpu.VMEM_SHARED((n_acc, d), f32),
            pltpu.VMEM_SHARED((n_acc,), f32),
            pltpu.VMEM((rpt,), f32),
            pltpu.VMEM((n_acc,), f32),
            pltpu.VMEM((2, 2, blk), jnp.int32),
            pltpu.VMEM((2, 2, blk), jnp.int32),
            pltpu.VMEM((2, 2, blk), f32),
            pltpu.VMEM((2, 2, blk), f32),
            [pltpu.VMEM((blk, d), f32)] * 2,
            pltpu.SemaphoreType.DMA,
            [pltpu.SemaphoreType.DMA] * 2,
            [pltpu.SemaphoreType.DMA] * 2,
        ],
    )
    def scat(x_hbm, src_hbm, dst_hbm, p_hbm, dparts_hbm,
             parts_hbm,
             acc_sh, den_sh, strip_buf, den_v,
             sidx3, didx3, p_blk3, alpha3, rows2, isem, gsem2, ssem2):
        rows_buf = rows2[0]
        c = lax.axis_index("c")
        s = lax.axis_index("s")
        wid = s * NC + c
        colbase = s * rpt

        def dzero(k, carry):
            den_v[pl.ds(colbase + k * L, L)] = jnp.zeros((L,), f32)
            return carry

        lax.fori_loop(0, rpt // L, dzero, 0)

        def dpart(r, carry):
            pltpu.sync_copy(dparts_hbm.at[r, pl.ds(colbase, rpt)], strip_buf)

            def dadd(k, carry2):
                sl = pl.ds(colbase + k * L, L)
                den_v[sl] = den_v[sl] + strip_buf[pl.ds(k * L, L)]
                return carry2

            lax.fori_loop(0, rpt // L, dadd, 0)
            return carry

        lax.fori_loop(0, NW, dpart, 0)
        pltpu.sync_copy(den_v.at[pl.ds(colbase, rpt)],
                        den_sh.at[pl.ds(colbase, rpt)])

        zer = jnp.zeros((L,), f32)

        def zrow(i, carry):
            for r in range(d // L):
                rows_buf[i, pl.ds(r * L, L)] = zer
            return carry

        lax.fori_loop(0, blk, zrow, 0)

        def zacc(j, carry):
            pltpu.sync_copy(rows_buf, acc_sh.at[pl.ds(colbase + j * blk, blk)])
            return carry

        lax.fori_loop(0, rpt // blk, zacc, 0)
        plsc.subcore_barrier()
        pltpu.sync_copy(den_sh, den_v)

        npair_i = tpair // NW + jnp.where(wid < tpair % NW, 1, 0)

        def fetch_idx(pj, par, start):
            for h in range(2):
                bb = pj * 2 * blk + h * blk
                for s_ref, d_ref in (
                        (src_hbm.at[pl.ds(bb, blk)], sidx3.at[par, h]),
                        (dst_hbm.at[pl.ds(bb, blk)], didx3.at[par, h]),
                        (p_hbm.at[pl.ds(bb, blk)], p_blk3.at[par, h])):
                    dsc = pltpu.make_async_copy(s_ref, d_ref, isem)
                    if start:
                        dsc.start()
                    else:
                        dsc.wait()

        def calc_alpha(par, h):
            def abody(k, carry2):
                sl = pl.ds(k * L, L)
                den16 = plsc.load_gather(den_v, [sidx3[par, h, sl]])
                alpha3[par, h, sl] = (p_blk3[par, h, sl]
                                      / jnp.maximum(den16, 1e-12))
                return carry2

            lax.fori_loop(0, blk // L, abody, 0)

        def scale_rows(par, h):
            base16 = jnp.full((L,), 0, jnp.int32)

            def sbody(ee, carry2):
                a16 = plsc.load_gather(
                    alpha3, [par + base16, h + base16,
                             jnp.full((L,), ee, jnp.int32)])
                for r in range(d // L):
                    sl = pl.ds(r * L, L)
                    rows2[h][ee, sl] = rows2[h][ee, sl] * a16
                return carry2

            lax.fori_loop(0, blk, sbody, 0)

        def drain_scatter(h):
            pltpu.make_async_copy(rows2[h], acc_sh.at[sidx3.at[0, h]],
                                  ssem2[h]).wait()

        fetch_idx(wid, 0, True)

        def mbody(jj, carry):
            par = lax.rem(jj, 2)
            pj = wid + jj * NW
            fetch_idx(pj, par, False)

            @pl.when(jj + 1 < npair_i)
            def _():
                fetch_idx(pj + NW, 1 - par, True)

            calc_alpha(par, 0)

            @pl.when(jj > 0)
            def _():
                drain_scatter(0)

            ga = pltpu.async_copy(x_hbm.at[didx3.at[par, 0]], rows2[0],
                                  gsem2[0])
            calc_alpha(par, 1)

            @pl.when(jj > 0)
            def _():
                drain_scatter(1)

            gb = pltpu.async_copy(x_hbm.at[didx3.at[par, 1]], rows2[1],
                                  gsem2[1])
            ga.wait()
            scale_rows(par, 0)
            pltpu.async_copy(rows2[0], acc_sh.at[sidx3.at[par, 0]], ssem2[0],
                             add=True)
            gb.wait()
            scale_rows(par, 1)
            pltpu.async_copy(rows2[1], acc_sh.at[sidx3.at[par, 1]], ssem2[1],
                             add=True)
            return carry

        lax.fori_loop(0, npair_i, mbody, 0)
        drain_scatter(0)
        drain_scatter(1)
        plsc.subcore_barrier()

        def dmp(j, carry):
            rb = colbase + j * blk
            pltpu.sync_copy(acc_sh.at[pl.ds(rb, blk)], rows_buf)
            pltpu.sync_copy(rows_buf, parts_hbm.at[c, pl.ds(rb, blk)])
            return carry

        lax.fori_loop(0, rpt // blk, dmp, 0)

    return scat


@functools.lru_cache(maxsize=None)
def _make_comb(n_pad, d, rows):
    def body(p_ref, o_ref):
        v = p_ref[0] + p_ref[1]
        o_ref[...] = jnp.where(v > 0, v, jnp.exp(jnp.minimum(v, 0.0)) - 1.0)

    return pl.pallas_call(
        body,
        grid=(n_pad // rows,),
        in_specs=[pl.BlockSpec((NC, rows, d), lambda i: (0, i, 0))],
        out_specs=pl.BlockSpec((rows, d), lambda i: (i, 0)),
        out_shape=jax.ShapeDtypeStruct((n_pad, d), f32),
    )


def kernel(x, user_retweet_message_times, poc_att, edge_index, t_o):
    n, d = x.shape
    e = edge_index.shape[1]
    n_pad = ((n + NW * 128 - 1) // (NW * 128)) * (NW * 128)

    xt_pad = jnp.pad(x.T, ((0, 0), (0, n_pad - n)))
    a1 = poc_att[0, :d]
    a2 = poc_att[0, d:]
    t_vec = jnp.full((L,), t_o, dtype=f32)
    src = edge_index[0]
    dst = edge_index[1]
    tt = user_retweet_message_times.astype(f32)

    n_acc = ((n + NW * L - 1) // (NW * L)) * (NW * L)

    s1, s2 = _make_proj(n_pad, d)(xt_pad, a1, a2)
    p, dparts = _make_edge(e, n_pad, n_acc)(src, dst, tt, s1, s2, t_vec)
    parts = _make_scat(e, n, n_acc, d)(x, src, dst, p, dparts)
    out = _make_comb(n_acc, d, n_acc // 16)(parts)
    return out[:n]


# speedup vs baseline: 26.8127x; 1.0554x over previous
import functools

import jax
import jax.numpy as jnp
from jax import lax
from jax.experimental import pallas as pl
from jax.experimental.pallas import tpu as pltpu
from jax.experimental.pallas import tpu_sc as plsc

NC = 2
NS = 16
L = 16
NW = NC * NS

f32 = jnp.float32


def _mesh():
    return plsc.VectorSubcoreMesh(core_axis_name="c", subcore_axis_name="s",
                                  num_cores=NC, num_subcores=NS)


@functools.lru_cache(maxsize=None)
def _make_proj(n_pad, d):
    npt = n_pad // NW

    @functools.partial(
        pl.kernel,
        out_type=[jax.ShapeDtypeStruct((n_pad,), f32),
                  jax.ShapeDtypeStruct((n_pad,), f32)],
        mesh=_mesh(),
        compiler_params=pltpu.CompilerParams(needs_layout_passes=False),
        scratch_types=[
            pltpu.VMEM((d, npt), f32),
            pltpu.VMEM((d,), f32),
            pltpu.VMEM((d,), f32),
            pltpu.VMEM((npt,), f32),
            pltpu.VMEM((npt,), f32),
        ],
    )
    def proj(xt_hbm, a1_hbm, a2_hbm, s1_hbm, s2_hbm, xt_v, a1_v, a2_v, s1_v, s2_v):
        wid = lax.axis_index("s") * NC + lax.axis_index("c")
        base = wid * npt
        pltpu.sync_copy(xt_hbm.at[:, pl.ds(base, npt)], xt_v)
        pltpu.sync_copy(a1_hbm, a1_v)
        pltpu.sync_copy(a2_hbm, a2_v)
        a1c = [a1_v[pl.ds(r * L, L)] for r in range(d // L)]
        a2c = [a2_v[pl.ds(r * L, L)] for r in range(d // L)]

        def body(g, carry):
            sl = pl.ds(g * L, L)
            acc1 = jnp.zeros((L,), f32)
            acc2 = jnp.zeros((L,), f32)
            for dd in range(d):
                col = xt_v[dd, sl]
                acc1 = acc1 + col * a1c[dd // L][dd % L]
                acc2 = acc2 + col * a2c[dd // L][dd % L]
            s1_v[sl] = acc1
            s2_v[sl] = acc2
            return carry

        lax.fori_loop(0, npt // L, body, 0)
        pltpu.sync_copy(s1_v, s1_hbm.at[pl.ds(base, npt)])
        pltpu.sync_copy(s2_v, s2_hbm.at[pl.ds(base, npt)])

    return proj


@functools.lru_cache(maxsize=None)
def _make_edge(e, n_pad, n_acc):
    ept = e // NW

    @functools.partial(
        pl.kernel,
        out_type=[jax.ShapeDtypeStruct((e,), f32),
                  jax.ShapeDtypeStruct((NW, n_acc), f32)],
        mesh=_mesh(),
        compiler_params=pltpu.CompilerParams(needs_layout_passes=False),
        scratch_types=[
            pltpu.VMEM((ept,), jnp.int32),
            pltpu.VMEM((ept,), jnp.int32),
            pltpu.VMEM((ept,), f32),
            pltpu.VMEM((ept,), f32),
            pltpu.VMEM((n_pad,), f32),
            pltpu.VMEM((n_pad,), f32),
            pltpu.VMEM((n_acc,), f32),
            pltpu.VMEM((L,), f32),
        ],
    )
    def edge(src_hbm, dst_hbm, tt_hbm, s1_hbm, s2_hbm, t_hbm,
             p_hbm, dparts_hbm,
             src_v, dst_v, tt_v, p_v, s1_v, s2_v, den_v, t_v):
        wid = lax.axis_index("s") * NC + lax.axis_index("c")
        base = wid * ept
        pltpu.sync_copy(src_hbm.at[pl.ds(base, ept)], src_v)
        pltpu.sync_copy(dst_hbm.at[pl.ds(base, ept)], dst_v)
        pltpu.sync_copy(tt_hbm.at[pl.ds(base, ept)], tt_v)
        pltpu.sync_copy(s1_hbm, s1_v)
        pltpu.sync_copy(s2_hbm, s2_v)
        pltpu.sync_copy(t_hbm, t_v)

        zer = jnp.zeros((L,), f32)

        def zbody(i, carry):
            den_v[pl.ds(i * L, L)] = zer
            return carry

        lax.fori_loop(0, n_acc // L, zbody, 0)

        t_vec = t_v[...]

        def ebody(i, carry):
            sl = pl.ds(i * L, L)
            si = src_v[sl]
            di = dst_v[sl]
            tt = tt_v[sl]
            g = plsc.load_gather(s1_v, [si]) + plsc.load_gather(s2_v, [di])
            g = jnp.where(g > 0, g, 0.01 * g)
            val = g * jnp.exp(-2.0 * (t_vec - tt))
            pe = jnp.exp(-val)
            p_v[sl] = pe
            plsc.addupdate_scatter(den_v, [si], pe)
            return carry

        lax.fori_loop(0, ept // L, ebody, 0)
        pltpu.sync_copy(p_v, p_hbm.at[pl.ds(base, ept)])
        pltpu.sync_copy(den_v, dparts_hbm.at[wid])

    return edge


@functools.lru_cache(maxsize=None)
def _make_scat(e, n, n_acc, d):
    blk = 128
    tblk = e // blk
    assert e % (2 * blk) == 0
    tpair = tblk // 2
    rpt = n_acc // NS
    assert rpt % blk == 0

    @functools.partial(
        pl.kernel,
        out_type=jax.ShapeDtypeStruct((NC, n_acc, d), f32),
        mesh=_mesh(),
        compiler_params=pltpu.CompilerParams(needs_layout_passes=False),
        scratch_types=[
            pltpu.VMEM_SHARED((n_acc, d), f32),
            pltpu.VMEM_SHARED((n_acc,), f32),
            pltpu.VMEM((2, rpt), f32),
            pltpu.VMEM((n_acc,), f32),
            pltpu.VMEM((2, 2, blk), jnp.int32),
            pltpu.VMEM((2, 2, blk), jnp.int32),
            pltpu.VMEM((2, 2, blk), f32),
            pltpu.VMEM((2, 2, blk), f32),
            [pltpu.VMEM((blk, d), f32)] * 2,
            pltpu.SemaphoreType.DMA,
            [pltpu.SemaphoreType.DMA] * 2,
            [pltpu.SemaphoreType.DMA] * 2,
        ],
    )
    def scat(x_hbm, src_hbm, dst_hbm, p_hbm, dparts_hbm,
             parts_hbm,
             acc_sh, den_sh, strip_buf, den_v,
             sidx3, didx3, p_blk3, alpha3, rows2, isem, gsem2, ssem2):
        rows_buf = rows2[0]
        c = lax.axis_index("c")
        s = lax.axis_index("s")
        wid = s * NC + c
        colbase = s * rpt

        def dzero(k, carry):
            den_v[pl.ds(colbase + k * L, L)] = jnp.zeros((L,), f32)
            return carry

        lax.fori_loop(0, rpt // L, dzero, 0)

        def strip_copy(r, par):
            return pltpu.make_async_copy(
                dparts_hbm.at[r, pl.ds(colbase, rpt)], strip_buf.at[par],
                isem)

        strip_copy(0, 0).start()

        def dpart(r, carry):
            par = lax.rem(r, 2)
            strip_copy(r, par).wait()

            @pl.when(r + 1 < NW)
            def _():
                strip_copy(r + 1, 1 - par).start()

            def dadd(k, carry2):
                sl = pl.ds(colbase + k * L, L)
                den_v[sl] = den_v[sl] + strip_buf[par, pl.ds(k * L, L)]
                return carry2

            lax.fori_loop(0, rpt // L, dadd, 0)
            return carry

        lax.fori_loop(0, NW, dpart, 0)
        pltpu.sync_copy(den_v.at[pl.ds(colbase, rpt)],
                        den_sh.at[pl.ds(colbase, rpt)])

        zer = jnp.zeros((L,), f32)

        def zrow(i, carry):
            for r in range(d // L):
                rows_buf[i, pl.ds(r * L, L)] = zer
            return carry

        lax.fori_loop(0, blk, zrow, 0)

        def zacc(j, carry):
            pltpu.sync_copy(rows_buf, acc_sh.at[pl.ds(colbase + j * blk, blk)])
            return carry

        lax.fori_loop(0, rpt // blk, zacc, 0)
        plsc.subcore_barrier()
        pltpu.sync_copy(den_sh, den_v)

        npair_i = tpair // NW + jnp.where(wid < tpair % NW, 1, 0)

        def fetch_idx(pj, par, start):
            for h in range(2):
                bb = pj * 2 * blk + h * blk
                for s_ref, d_ref in (
                        (src_hbm.at[pl.ds(bb, blk)], sidx3.at[par, h]),
                        (dst_hbm.at[pl.ds(bb, blk)], didx3.at[par, h]),
                        (p_hbm.at[pl.ds(bb, blk)], p_blk3.at[par, h])):
                    dsc = pltpu.make_async_copy(s_ref, d_ref, isem)
                    if start:
                        dsc.start()
                    else:
                        dsc.wait()

        def calc_alpha(par, h):
            def abody(k, carry2):
                sl = pl.ds(k * L, L)
                den16 = plsc.load_gather(den_v, [sidx3[par, h, sl]])
                alpha3[par, h, sl] = (p_blk3[par, h, sl]
                                      / jnp.maximum(den16, 1e-12))
                return carry2

            lax.fori_loop(0, blk // L, abody, 0)

        def scale_rows(par, h):
            base16 = jnp.full((L,), 0, jnp.int32)

            def sbody(eh, carry2):
                for u in range(2):
                    ee = eh * 2 + u
                    a16 = plsc.load_gather(
                        alpha3, [par + base16, h + base16,
                                 jnp.full((L,), ee, jnp.int32)])
                    for r in range(d // L):
                        sl = pl.ds(r * L, L)
                        rows2[h][ee, sl] = rows2[h][ee, sl] * a16
                return carry2

            lax.fori_loop(0, blk // 2, sbody, 0)

        def drain_scatter(h):
            pltpu.make_async_copy(rows2[h], acc_sh.at[sidx3.at[0, h]],
                                  ssem2[h]).wait()

        fetch_idx(wid, 0, True)

        def mbody(jj, carry):
            par = lax.rem(jj, 2)
            pj = wid + jj * NW
            fetch_idx(pj, par, False)

            @pl.when(jj + 1 < npair_i)
            def _():
                fetch_idx(pj + NW, 1 - par, True)

            calc_alpha(par, 0)

            @pl.when(jj > 0)
            def _():
                drain_scatter(0)

            ga = pltpu.async_copy(x_hbm.at[didx3.at[par, 0]], rows2[0],
                                  gsem2[0])
            calc_alpha(par, 1)

            @pl.when(jj > 0)
            def _():
                drain_scatter(1)

            gb = pltpu.async_copy(x_hbm.at[didx3.at[par, 1]], rows2[1],
                                  gsem2[1])
            ga.wait()
            scale_rows(par, 0)
            pltpu.async_copy(rows2[0], acc_sh.at[sidx3.at[par, 0]], ssem2[0],
                             add=True)
            gb.wait()
            scale_rows(par, 1)
            pltpu.async_copy(rows2[1], acc_sh.at[sidx3.at[par, 1]], ssem2[1],
                             add=True)
            return carry

        lax.fori_loop(0, npair_i, mbody, 0)
        drain_scatter(0)
        drain_scatter(1)
        plsc.subcore_barrier()

        def dmp(j, carry):
            rb = colbase + j * blk
            pltpu.sync_copy(acc_sh.at[pl.ds(rb, blk)], rows_buf)
            pltpu.sync_copy(rows_buf, parts_hbm.at[c, pl.ds(rb, blk)])
            return carry

        lax.fori_loop(0, rpt // blk, dmp, 0)

    return scat


@functools.lru_cache(maxsize=None)
def _make_comb(n_pad, d, rows):
    def body(p_ref, o_ref):
        v = p_ref[0] + p_ref[1]
        o_ref[...] = jnp.where(v > 0, v, jnp.exp(jnp.minimum(v, 0.0)) - 1.0)

    return pl.pallas_call(
        body,
        grid=(n_pad // rows,),
        in_specs=[pl.BlockSpec((NC, rows, d), lambda i: (0, i, 0))],
        out_specs=pl.BlockSpec((rows, d), lambda i: (i, 0)),
        out_shape=jax.ShapeDtypeStruct((n_pad, d), f32),
    )


def kernel(x, user_retweet_message_times, poc_att, edge_index, t_o):
    n, d = x.shape
    e = edge_index.shape[1]
    n_pad = ((n + NW * 128 - 1) // (NW * 128)) * (NW * 128)

    xt_pad = jnp.pad(x.T, ((0, 0), (0, n_pad - n)))
    a1 = poc_att[0, :d]
    a2 = poc_att[0, d:]
    t_vec = jnp.full((L,), t_o, dtype=f32)
    src = edge_index[0]
    dst = edge_index[1]
    tt = user_retweet_message_times.astype(f32)

    n_acc = ((n + NW * L - 1) // (NW * L)) * (NW * L)

    s1, s2 = _make_proj(n_pad, d)(xt_pad, a1, a2)
    p, dparts = _make_edge(e, n_pad, n_acc)(src, dst, tt, s1, s2, t_vec)
    parts = _make_scat(e, n, n_acc, d)(x, src, dst, p, dparts)
    out = _make_comb(n_acc, d, n_acc // 16)(parts)
    return out[:n]


# speedup vs baseline: 27.4151x; 1.0225x over previous
import functools

import jax
import jax.numpy as jnp
from jax import lax
from jax.experimental import pallas as pl
from jax.experimental.pallas import tpu as pltpu
from jax.experimental.pallas import tpu_sc as plsc

NC = 2
NS = 16
L = 16
NW = NC * NS

f32 = jnp.float32


def _mesh():
    return plsc.VectorSubcoreMesh(core_axis_name="c", subcore_axis_name="s",
                                  num_cores=NC, num_subcores=NS)


@functools.lru_cache(maxsize=None)
def _make_proj(n_pad, d):
    npt = n_pad // NW

    @functools.partial(
        pl.kernel,
        out_type=[jax.ShapeDtypeStruct((n_pad,), f32),
                  jax.ShapeDtypeStruct((n_pad,), f32)],
        mesh=_mesh(),
        compiler_params=pltpu.CompilerParams(needs_layout_passes=False),
        scratch_types=[
            pltpu.VMEM((d, npt), f32),
            pltpu.VMEM((d,), f32),
            pltpu.VMEM((d,), f32),
            pltpu.VMEM((npt,), f32),
            pltpu.VMEM((npt,), f32),
            pltpu.SemaphoreType.DMA,
        ],
    )
    def proj(xt_hbm, a1_hbm, a2_hbm, s1_hbm, s2_hbm, xt_v, a1_v, a2_v, s1_v,
             s2_v, sem):
        wid = lax.axis_index("s") * NC + lax.axis_index("c")
        base = wid * npt
        copies = [
            pltpu.make_async_copy(xt_hbm.at[:, pl.ds(base, npt)], xt_v, sem),
            pltpu.make_async_copy(a1_hbm, a1_v, sem),
            pltpu.make_async_copy(a2_hbm, a2_v, sem),
        ]
        for cp in copies:
            cp.start()
        for cp in copies:
            cp.wait()
        a1c = [a1_v[pl.ds(r * L, L)] for r in range(d // L)]
        a2c = [a2_v[pl.ds(r * L, L)] for r in range(d // L)]

        def body(g, carry):
            sl = pl.ds(g * L, L)
            acc1 = jnp.zeros((L,), f32)
            acc2 = jnp.zeros((L,), f32)
            for dd in range(d):
                col = xt_v[dd, sl]
                acc1 = acc1 + col * a1c[dd // L][dd % L]
                acc2 = acc2 + col * a2c[dd // L][dd % L]
            s1_v[sl] = acc1
            s2_v[sl] = acc2
            return carry

        lax.fori_loop(0, npt // L, body, 0)
        pltpu.sync_copy(s1_v, s1_hbm.at[pl.ds(base, npt)])
        pltpu.sync_copy(s2_v, s2_hbm.at[pl.ds(base, npt)])

    return proj


@functools.lru_cache(maxsize=None)
def _make_edge(e, n_pad, n_acc):
    ept = e // NW

    @functools.partial(
        pl.kernel,
        out_type=[jax.ShapeDtypeStruct((e,), f32),
                  jax.ShapeDtypeStruct((NW, n_acc), f32)],
        mesh=_mesh(),
        compiler_params=pltpu.CompilerParams(needs_layout_passes=False),
        scratch_types=[
            pltpu.VMEM((ept,), jnp.int32),
            pltpu.VMEM((ept,), jnp.int32),
            pltpu.VMEM((ept,), f32),
            pltpu.VMEM((ept,), f32),
            pltpu.VMEM((n_pad,), f32),
            pltpu.VMEM((n_pad,), f32),
            pltpu.VMEM((n_acc,), f32),
            pltpu.VMEM((L,), f32),
            pltpu.SemaphoreType.DMA,
        ],
    )
    def edge(src_hbm, dst_hbm, tt_hbm, s1_hbm, s2_hbm, t_hbm,
             p_hbm, dparts_hbm,
             src_v, dst_v, tt_v, p_v, s1_v, s2_v, den_v, t_v, sem):
        wid = lax.axis_index("s") * NC + lax.axis_index("c")
        base = wid * ept
        copies = [
            pltpu.make_async_copy(src_hbm.at[pl.ds(base, ept)], src_v, sem),
            pltpu.make_async_copy(dst_hbm.at[pl.ds(base, ept)], dst_v, sem),
            pltpu.make_async_copy(tt_hbm.at[pl.ds(base, ept)], tt_v, sem),
            pltpu.make_async_copy(s1_hbm, s1_v, sem),
            pltpu.make_async_copy(s2_hbm, s2_v, sem),
            pltpu.make_async_copy(t_hbm, t_v, sem),
        ]
        for cp in copies:
            cp.start()

        zer = jnp.zeros((L,), f32)

        def zbody(i, carry):
            den_v[pl.ds(i * L, L)] = zer
            return carry

        lax.fori_loop(0, n_acc // L, zbody, 0)
        for cp in copies:
            cp.wait()

        t_vec = t_v[...]

        def ebody(i, carry):
            sl = pl.ds(i * L, L)
            si = src_v[sl]
            di = dst_v[sl]
            tt = tt_v[sl]
            g = plsc.load_gather(s1_v, [si]) + plsc.load_gather(s2_v, [di])
            g = jnp.where(g > 0, g, 0.01 * g)
            val = g * jnp.exp(-2.0 * (t_vec - tt))
            pe = jnp.exp(-val)
            p_v[sl] = pe
            plsc.addupdate_scatter(den_v, [si], pe)
            return carry

        lax.fori_loop(0, ept // L, ebody, 0)
        pltpu.sync_copy(p_v, p_hbm.at[pl.ds(base, ept)])
        pltpu.sync_copy(den_v, dparts_hbm.at[wid])

    return edge


@functools.lru_cache(maxsize=None)
def _make_scat(e, n, n_acc, d):
    blk = 128
    tblk = e // blk
    assert e % (2 * blk) == 0
    tpair = tblk // 2
    rpt = n_acc // NS
    assert rpt % blk == 0

    @functools.partial(
        pl.kernel,
        out_type=jax.ShapeDtypeStruct((NC, n_acc, d), f32),
        mesh=_mesh(),
        compiler_params=pltpu.CompilerParams(needs_layout_passes=False),
        scratch_types=[
            pltpu.VMEM_SHARED((n_acc, d), f32),
            pltpu.VMEM_SHARED((n_acc,), f32),
            pltpu.VMEM((2, rpt), f32),
            pltpu.VMEM((n_acc,), f32),
            pltpu.VMEM((2, 2, blk), jnp.int32),
            pltpu.VMEM((2, 2, blk), jnp.int32),
            pltpu.VMEM((2, 2, blk), f32),
            pltpu.VMEM((2, 2, blk), f32),
            [pltpu.VMEM((blk, d), f32)] * 2,
            pltpu.SemaphoreType.DMA,
            [pltpu.SemaphoreType.DMA] * 2,
            [pltpu.SemaphoreType.DMA] * 2,
        ],
    )
    def scat(x_hbm, src_hbm, dst_hbm, p_hbm, dparts_hbm,
             parts_hbm,
             acc_sh, den_sh, strip_buf, den_v,
             sidx3, didx3, p_blk3, alpha3, rows2, isem, gsem2, ssem2):
        rows_buf = rows2[0]
        c = lax.axis_index("c")
        s = lax.axis_index("s")
        wid = s * NC + c
        colbase = s * rpt

        def dzero(k, carry):
            den_v[pl.ds(colbase + k * L, L)] = jnp.zeros((L,), f32)
            return carry

        lax.fori_loop(0, rpt // L, dzero, 0)

        def strip_copy(r, par):
            return pltpu.make_async_copy(
                dparts_hbm.at[r, pl.ds(colbase, rpt)], strip_buf.at[par],
                isem)

        strip_copy(0, 0).start()

        def dpart(r, carry):
            par = lax.rem(r, 2)
            strip_copy(r, par).wait()

            @pl.when(r + 1 < NW)
            def _():
                strip_copy(r + 1, 1 - par).start()

            def dadd(k, carry2):
                sl = pl.ds(colbase + k * L, L)
                den_v[sl] = den_v[sl] + strip_buf[par, pl.ds(k * L, L)]
                return carry2

            lax.fori_loop(0, rpt // L, dadd, 0)
            return carry

        lax.fori_loop(0, NW, dpart, 0)
        pltpu.sync_copy(den_v.at[pl.ds(colbase, rpt)],
                        den_sh.at[pl.ds(colbase, rpt)])

        zer = jnp.zeros((L,), f32)

        def zrow(i, carry):
            for r in range(d // L):
                rows_buf[i, pl.ds(r * L, L)] = zer
            return carry

        lax.fori_loop(0, blk, zrow, 0)

        def zacc(j, carry):
            pltpu.sync_copy(rows_buf, acc_sh.at[pl.ds(colbase + j * blk, blk)])
            return carry

        lax.fori_loop(0, rpt // blk, zacc, 0)
        plsc.subcore_barrier()
        pltpu.sync_copy(den_sh, den_v)

        npair_i = tpair // NW + jnp.where(wid < tpair % NW, 1, 0)

        def fetch_idx(pj, par, start):
            for h in range(2):
                bb = pj * 2 * blk + h * blk
                for s_ref, d_ref in (
                        (src_hbm.at[pl.ds(bb, blk)], sidx3.at[par, h]),
                        (dst_hbm.at[pl.ds(bb, blk)], didx3.at[par, h]),
                        (p_hbm.at[pl.ds(bb, blk)], p_blk3.at[par, h])):
                    dsc = pltpu.make_async_copy(s_ref, d_ref, isem)
                    if start:
                        dsc.start()
                    else:
                        dsc.wait()

        def calc_alpha(par, h):
            def abody(k, carry2):
                sl = pl.ds(k * L, L)
                den16 = plsc.load_gather(den_v, [sidx3[par, h, sl]])
                alpha3[par, h, sl] = (p_blk3[par, h, sl]
                                      / jnp.maximum(den16, 1e-12))
                return carry2

            lax.fori_loop(0, blk // L, abody, 0)

        def scale_rows(par, h):
            base16 = jnp.full((L,), 0, jnp.int32)

            def sbody(eh, carry2):
                for u in range(2):
                    ee = eh * 2 + u
                    a16 = plsc.load_gather(
                        alpha3, [par + base16, h + base16,
                                 jnp.full((L,), ee, jnp.int32)])
                    for r in range(d // L):
                        sl = pl.ds(r * L, L)
                        rows2[h][ee, sl] = rows2[h][ee, sl] * a16
                return carry2

            lax.fori_loop(0, blk // 2, sbody, 0)

        def drain_scatter(h):
            pltpu.make_async_copy(rows2[h], acc_sh.at[sidx3.at[0, h]],
                                  ssem2[h]).wait()

        fetch_idx(wid, 0, True)

        def mbody(jj, carry):
            par = lax.rem(jj, 2)
            pj = wid + jj * NW
            fetch_idx(pj, par, False)

            @pl.when(jj + 1 < npair_i)
            def _():
                fetch_idx(pj + NW, 1 - par, True)

            calc_alpha(par, 0)

            @pl.when(jj > 0)
            def _():
                drain_scatter(0)

            ga = pltpu.async_copy(x_hbm.at[didx3.at[par, 0]], rows2[0],
                                  gsem2[0])
            calc_alpha(par, 1)

            @pl.when(jj > 0)
            def _():
                drain_scatter(1)

            gb = pltpu.async_copy(x_hbm.at[didx3.at[par, 1]], rows2[1],
                                  gsem2[1])
            ga.wait()
            scale_rows(par, 0)
            pltpu.async_copy(rows2[0], acc_sh.at[sidx3.at[par, 0]], ssem2[0],
                             add=True)
            gb.wait()
            scale_rows(par, 1)
            pltpu.async_copy(rows2[1], acc_sh.at[sidx3.at[par, 1]], ssem2[1],
                             add=True)
            return carry

        lax.fori_loop(0, npair_i, mbody, 0)
        drain_scatter(0)
        drain_scatter(1)
        plsc.subcore_barrier()

        def dmp(j, carry):
            rb = colbase + j * blk
            pltpu.sync_copy(acc_sh.at[pl.ds(rb, blk)], rows_buf)
            pltpu.sync_copy(rows_buf, parts_hbm.at[c, pl.ds(rb, blk)])
            return carry

        lax.fori_loop(0, rpt // blk, dmp, 0)

    return scat


@functools.lru_cache(maxsize=None)
def _make_comb(n_pad, d, rows):
    def body(p_ref, o_ref):
        v = p_ref[0] + p_ref[1]
        o_ref[...] = jnp.where(v > 0, v, jnp.exp(jnp.minimum(v, 0.0)) - 1.0)

    return pl.pallas_call(
        body,
        grid=(n_pad // rows,),
        in_specs=[pl.BlockSpec((NC, rows, d), lambda i: (0, i, 0))],
        out_specs=pl.BlockSpec((rows, d), lambda i: (i, 0)),
        out_shape=jax.ShapeDtypeStruct((n_pad, d), f32),
    )


def kernel(x, user_retweet_message_times, poc_att, edge_index, t_o):
    n, d = x.shape
    e = edge_index.shape[1]
    n_pad = ((n + NW * 128 - 1) // (NW * 128)) * (NW * 128)

    xt_pad = jnp.pad(x.T, ((0, 0), (0, n_pad - n)))
    a1 = poc_att[0, :d]
    a2 = poc_att[0, d:]
    t_vec = jnp.full((L,), t_o, dtype=f32)
    src = edge_index[0]
    dst = edge_index[1]
    tt = user_retweet_message_times.astype(f32)

    n_acc = ((n + NW * L - 1) // (NW * L)) * (NW * L)

    s1, s2 = _make_proj(n_pad, d)(xt_pad, a1, a2)
    p, dparts = _make_edge(e, n_pad, n_acc)(src, dst, tt, s1, s2, t_vec)
    parts = _make_scat(e, n, n_acc, d)(x, src, dst, p, dparts)
    out = _make_comb(n_acc, d, n_acc // 16)(parts)
    return out[:n]
